# baseline (speedup 1.0000x reference)
import functools

import jax
import jax.numpy as jnp
from jax import lax
from jax.experimental import pallas as pl
from jax.experimental.pallas import tpu as pltpu
from jax.experimental.pallas import tpu_sc as plsc

_MAXLEN = 200
_EMBED = 64
_BATCH = 4096

_NW = 32
_BL = _BATCH * _MAXLEN
_ROWS_PER_W = _BL // _NW
_SEQ_PER_W = _ROWS_PER_W // _MAXLEN
_G0 = 104
_G1 = _MAXLEN - _G0
_NSL = _EMBED // 16


def _body(x_hbm, tok_hbm, pos_hbm, out_hbm, idx_v, rows_v, pos_v, sem):
    wid = lax.axis_index("s") * 2 + lax.axis_index("c")
    base0 = wid * _ROWS_PER_W

    pltpu.sync_copy(pos_hbm, pos_v)

    def chunk(c, carry):
        base = base0 + c * _MAXLEN
        pltpu.sync_copy(x_hbm.at[pl.ds(base, _MAXLEN)], idx_v)
        cp0 = pltpu.async_copy(
            tok_hbm.at[idx_v.at[pl.ds(0, _G0)]],
            rows_v.at[pl.ds(0, _G0)], sem)
        cp1 = pltpu.async_copy(
            tok_hbm.at[idx_v.at[pl.ds(_G0, _G1)]],
            rows_v.at[pl.ds(_G0, _G1)], sem)
        cp0.wait()
        cp1.wait()

        def add_row(l, carry2):
            for j in range(_NSL):
                sl = pl.ds(j * 16, 16)
                rows_v[l, sl] = rows_v[l, sl] + pos_v[l, sl]
            return carry2

        lax.fori_loop(0, _MAXLEN, add_row, 0, unroll=2)
        pltpu.sync_copy(rows_v, out_hbm.at[pl.ds(base, _MAXLEN)])
        return carry

    lax.fori_loop(0, _SEQ_PER_W, chunk, 0)


def kernel(x, token_table, pos_table):
    B, L = x.shape
    E = token_table.shape[1]
    x_flat = x.reshape(B * L).astype(jnp.int32)

    k = pl.kernel(
        _body,
        out_type=jax.ShapeDtypeStruct((B * L, E), jnp.float32),
        mesh=plsc.VectorSubcoreMesh(core_axis_name="c", subcore_axis_name="s"),
        scratch_types=[
            pltpu.VMEM((_MAXLEN,), jnp.int32),
            pltpu.VMEM((_MAXLEN, _EMBED), jnp.float32),
            pltpu.VMEM((_MAXLEN, _EMBED), jnp.float32),
            pltpu.SemaphoreType.DMA,
        ],
        compiler_params=pltpu.CompilerParams(use_tc_tiling_on_sc=False),
    )
    out = k(x_flat, token_table, pos_table)
    return out.reshape(B, L, E)


# speedup vs baseline: 1.4433x; 1.4433x over previous
import jax
import jax.numpy as jnp
from jax import lax
from jax.experimental import pallas as pl
from jax.experimental.pallas import tpu as pltpu
from jax.experimental.pallas import tpu_sc as plsc

_MAXLEN = 200
_EMBED = 64
_BATCH = 4096

_NW = 32
_BL = _BATCH * _MAXLEN
_ROWS_PER_W = _BL // _NW
_NCHUNK = _ROWS_PER_W // _MAXLEN
_G0 = 104
_G1 = _MAXLEN - _G0
_NSL = _EMBED // 16


def _body(x_hbm, tok_hbm, pos_hbm, out_hbm,
          idx_all, rows0, rows1, pos_v, sem_g0, sem_g1, sem_w0, sem_w1):
    wid = lax.axis_index("s") * 2 + lax.axis_index("c")
    base0 = wid * _ROWS_PER_W

    pltpu.sync_copy(x_hbm.at[pl.ds(base0, _ROWS_PER_W)], idx_all)
    pltpu.sync_copy(pos_hbm, pos_v)

    rows = (rows0, rows1)
    sem_g = (sem_g0, sem_g1)
    sem_w = (sem_w0, sem_w1)

    def start_gather(c, buf, sem):
        off = pl.multiple_of(c * _MAXLEN, 8)
        pltpu.async_copy(tok_hbm.at[idx_all.at[pl.ds(off, _G0)]],
                         buf.at[pl.ds(0, _G0)], sem)
        pltpu.async_copy(tok_hbm.at[idx_all.at[pl.ds(off + _G0, _G1)]],
                         buf.at[pl.ds(_G0, _G1)], sem)

    def wait_gather(buf, sem):
        pltpu.make_async_copy(out_hbm.at[pl.ds(0, _MAXLEN)], buf, sem).wait()

    def start_write(c, buf, sem):
        pltpu.async_copy(buf, out_hbm.at[pl.ds(base0 + c * _MAXLEN, _MAXLEN)],
                         sem)

    def wait_write(buf, sem):
        pltpu.make_async_copy(buf, out_hbm.at[pl.ds(0, _MAXLEN)], sem).wait()

    start_gather(0, rows0, sem_g0)

    def outer(cc, carry):
        for b in range(2):
            c = cc * 2 + b
            cur, nxt = rows[b], rows[1 - b]

            @pl.when(c >= 1)
            def _():
                wait_write(nxt, sem_w[1 - b])

            @pl.when(c + 1 < _NCHUNK)
            def _():
                start_gather(c + 1, nxt, sem_g[1 - b])

            wait_gather(cur, sem_g[b])

            def add_row(l, carry2):
                for j in range(_NSL):
                    sl = pl.ds(j * 16, 16)
                    plsc.addupdate(cur.at[l, sl], pos_v[l, sl])
                return carry2

            lax.fori_loop(0, _MAXLEN, add_row, 0, unroll=4)
            start_write(c, cur, sem_w[b])
        return carry

    lax.fori_loop(0, _NCHUNK // 2, outer, 0)
    wait_write(rows1, sem_w1)


def kernel(x, token_table, pos_table):
    B, L = x.shape
    E = token_table.shape[1]
    x_flat = x.reshape(B * L).astype(jnp.int32)

    k = pl.kernel(
        _body,
        out_type=jax.ShapeDtypeStruct((B * L, E), jnp.float32),
        mesh=plsc.VectorSubcoreMesh(core_axis_name="c", subcore_axis_name="s"),
        scratch_types=[
            pltpu.VMEM((_ROWS_PER_W,), jnp.int32),
            pltpu.VMEM((_MAXLEN, _EMBED), jnp.float32),
            pltpu.VMEM((_MAXLEN, _EMBED), jnp.float32),
            pltpu.VMEM((_MAXLEN, _EMBED), jnp.float32),
            pltpu.SemaphoreType.DMA,
            pltpu.SemaphoreType.DMA,
            pltpu.SemaphoreType.DMA,
            pltpu.SemaphoreType.DMA,
        ],
        compiler_params=pltpu.CompilerParams(use_tc_tiling_on_sc=False),
    )
    out = k(x_flat, token_table, pos_table)
    return out.reshape(B, L, E)
